# Initial kernel scaffold; baseline (speedup 1.0000x reference)
import jax
import jax.numpy as jnp
from jax.experimental import pallas as pl


def kernel(x, edge_index, W1, b1, W2, b2):
    raise NotImplementedError("write your pallas kernel here")


# speedup vs baseline: 10.7109x; 10.7109x over previous
import functools

import jax
import jax.numpy as jnp
from jax import lax
from jax.experimental import pallas as pl
from jax.experimental.pallas import tpu as pltpu
from jax.experimental.pallas import tpu_sc as plsc

N = 10000
E = 320000
D1 = 128
D2 = 16
NC = 2
NS = 16
NW = NC * NS
EW = 10240
EPAD = EW * NW
K = 128
NCH = EW // K
NACC = 10240
RPT = NACC // NS
BM = 1000


def _sc_mesh():
    return plsc.VectorSubcoreMesh(core_axis_name="c", subcore_axis_name="s")


def _make_agg(D):
    @functools.partial(
        pl.kernel,
        out_type=jax.ShapeDtypeStruct((NC, NACC, D), jnp.float32),
        mesh=_sc_mesh(),
        compiler_params=pltpu.CompilerParams(use_tc_tiling_on_sc=False),
        scratch_types=[
            pltpu.VMEM((K,), jnp.int32),
            pltpu.VMEM((K,), jnp.int32),
            pltpu.VMEM((K, D), jnp.float32),
            pltpu.VMEM_SHARED((NACC, D), jnp.float32),
            pltpu.SemaphoreType.DMA,
        ],
    )
    def agg(table, srcp, dstp, out, sidx, didx, rows, acc, sem):
        c = lax.axis_index("c")
        s = lax.axis_index("s")
        wid = s * NC + c

        def zrow(i, carry):
            for j in range(D // 16):
                rows[i, pl.ds(j * 16, 16)] = jnp.zeros((16,), jnp.float32)
            return carry

        lax.fori_loop(0, K, zrow, 0)
        r0 = s * RPT

        def zcp(t, carry):
            pltpu.sync_copy(rows, acc.at[pl.ds(r0 + t * K, K)])
            return carry

        lax.fori_loop(0, RPT // K, zcp, 0)
        plsc.subcore_barrier()

        base = wid * EW

        def step(j, carry):
            off = base + j * K
            pltpu.sync_copy(srcp.at[pl.ds(off, K)], sidx)
            pltpu.sync_copy(dstp.at[pl.ds(off, K)], didx)
            pltpu.async_copy(table.at[sidx], rows, sem).wait()
            pltpu.sync_copy(rows, acc.at[didx], add=True)
            return carry

        lax.fori_loop(0, NCH, step, 0)
        plsc.subcore_barrier()
        pltpu.sync_copy(acc.at[pl.ds(r0, RPT)], out.at[c, pl.ds(r0, RPT)])

    return agg


@functools.partial(
    pl.kernel,
    out_type=jax.ShapeDtypeStruct((NC, NACC), jnp.float32),
    mesh=_sc_mesh(),
    scratch_types=[
        pltpu.VMEM((K,), jnp.int32),
        pltpu.VMEM((K,), jnp.float32),
        pltpu.VMEM((RPT,), jnp.float32),
        pltpu.VMEM_SHARED((NACC,), jnp.float32),
    ],
)
def _deg_kernel(dstp, out, didx, ones, zbuf, acc):
    c = lax.axis_index("c")
    s = lax.axis_index("s")
    wid = s * NC + c
    for j in range(K // 16):
        ones[pl.ds(j * 16, 16)] = jnp.ones((16,), jnp.float32)

    def zfill(i, carry):
        zbuf[pl.ds(i * 16, 16)] = jnp.zeros((16,), jnp.float32)
        return carry

    lax.fori_loop(0, RPT // 16, zfill, 0)
    r0 = s * RPT
    pltpu.sync_copy(zbuf, acc.at[pl.ds(r0, RPT)])
    plsc.subcore_barrier()

    base = wid * EW

    def step(j, carry):
        pltpu.sync_copy(dstp.at[pl.ds(base + j * K, K)], didx)
        pltpu.sync_copy(ones, acc.at[didx], add=True)
        return carry

    lax.fori_loop(0, NCH, step, 0)
    plsc.subcore_barrier()
    pltpu.sync_copy(acc.at[pl.ds(r0, RPT)], out.at[c, pl.ds(r0, RPT)])


def _dinv_of(degb):
    d = degb[...]
    deg = d[:, 0:1] + d[:, 1:2] + 1.0
    return lax.rsqrt(deg)


def _dense1_body(xb, w, degb, ob):
    dinv = _dinv_of(degb)
    ob[...] = jnp.dot(xb[...], w[...], preferred_element_type=jnp.float32) * dinv


def _dense2_body(s1a, s1b, h1s, degb, b1, w2p, ob):
    dinv = _dinv_of(degb)
    out1 = dinv * (s1a[...] + s1b[...] + h1s[...])
    a1 = jnp.maximum(out1 + b1[...], 0.0)
    ob[...] = jnp.dot(a1, w2p[...], preferred_element_type=jnp.float32) * dinv


def _final_body(s2a, s2b, h2s, degb, b2, ob):
    dinv = _dinv_of(degb)
    out2 = dinv * (s2a[...] + s2b[...] + h2s[...])
    bv = b2[...]
    za = out2[:, 0:1] + bv[:, 0:1]
    zb = out2[:, 1:2] + bv[:, 1:2]
    m = jnp.maximum(za, zb)
    lse = m + jnp.log(jnp.exp(za - m) + jnp.exp(zb - m))
    ob[...] = jnp.concatenate([za - lse, zb - lse], axis=1)


def _row_spec(d):
    return pl.BlockSpec((BM, d), lambda i: (i, 0))


def _full_spec(shape):
    return pl.BlockSpec(shape, lambda i: (0, 0))


_dense1 = pl.pallas_call(
    _dense1_body,
    grid=(N // BM,),
    in_specs=[_row_spec(D1), _full_spec((D1, D1)), _row_spec(2)],
    out_specs=_row_spec(D1),
    out_shape=jax.ShapeDtypeStruct((N, D1), jnp.float32),
)

_dense2 = pl.pallas_call(
    _dense2_body,
    grid=(N // BM,),
    in_specs=[_row_spec(D1), _row_spec(D1), _row_spec(D1), _row_spec(2),
              _full_spec((1, D1)), _full_spec((D1, D2))],
    out_specs=_row_spec(D2),
    out_shape=jax.ShapeDtypeStruct((N, D2), jnp.float32),
)

_final = pl.pallas_call(
    _final_body,
    grid=(N // BM,),
    in_specs=[_row_spec(D2), _row_spec(D2), _row_spec(D2), _row_spec(2),
              _full_spec((1, 2))],
    out_specs=_row_spec(2),
    out_shape=jax.ShapeDtypeStruct((N, 2), jnp.float32),
)

_agg128 = _make_agg(D1)
_agg16 = _make_agg(D2)


def kernel(x, edge_index, W1, b1, W2, b2):
    src = edge_index[0].astype(jnp.int32)
    dst = edge_index[1].astype(jnp.int32)
    pad = EPAD - E
    srcp = jnp.concatenate([src, jnp.zeros((pad,), jnp.int32)])
    dstp = jnp.concatenate([dst, jnp.full((pad,), N, jnp.int32)])

    degp = _deg_kernel(dstp)
    degt = jnp.transpose(degp)[:N]

    h1s = _dense1(x, W1, degt)
    s1 = _agg128(h1s, srcp, dstp)

    W2p = jnp.concatenate(
        [W2, jnp.zeros((D1, D2 - W2.shape[1]), jnp.float32)], axis=1)
    h2s = _dense2(s1[0, :N], s1[1, :N], h1s, degt,
                  b1.reshape(1, D1), W2p)
    s2 = _agg16(h2s, srcp, dstp)

    out = _final(s2[0, :N], s2[1, :N], h2s, degt, b2.reshape(1, 2))
    return out


# speedup vs baseline: 16.0166x; 1.4953x over previous
import functools

import jax
import jax.numpy as jnp
from jax import lax
from jax.experimental import pallas as pl
from jax.experimental.pallas import tpu as pltpu
from jax.experimental.pallas import tpu_sc as plsc

N = 10000
E = 320000
D1 = 128
DH = 64
D2 = 16
NC = 2
NS = 16
NW = NC * NS
EW = 10240
EPAD = EW * NW
K = 128
NCH = EW // K
EWS = EPAD // NS
NPASS = 2
NCHS = EWS // (K * NPASS)
NACC = 10240
RPT = NACC // NS
BM = 1000


def _sc_mesh():
    return plsc.VectorSubcoreMesh(core_axis_name="c", subcore_axis_name="s")


@functools.partial(
    pl.kernel,
    out_type=jax.ShapeDtypeStruct((NACC, D1), jnp.float32),
    mesh=_sc_mesh(),
    compiler_params=pltpu.CompilerParams(use_tc_tiling_on_sc=False),
    scratch_types=[
        pltpu.VMEM((NCHS, K), jnp.int32),
        pltpu.VMEM((NCHS, K), jnp.int32),
        pltpu.VMEM((8 * K, DH), jnp.float32),
        pltpu.VMEM_SHARED((NACC, DH), jnp.float32),
        pltpu.SemaphoreType.DMA,
        pltpu.SemaphoreType.DMA,
    ],
)
def _agg128(table, src5, dst4, out, sidx, didx, rows, acc, semg, sems):
    NBUF = 8
    c = lax.axis_index("c")
    s = lax.axis_index("s")
    r0 = s * RPT

    def zrow(i, carry):
        for j in range(DH // 16):
            rows[i, pl.ds(j * 16, 16)] = jnp.zeros((16,), jnp.float32)
        return carry

    lax.fori_loop(0, K, zrow, 0)

    def zcp(t, carry):
        pltpu.sync_copy(rows.at[pl.ds(0, K)], acc.at[pl.ds(r0 + t * K, K)])
        return carry

    lax.fori_loop(0, RPT // K, zcp, 0)
    plsc.subcore_barrier()

    for p in range(NPASS):
        pltpu.sync_copy(src5.at[c, s, p], sidx)
        pltpu.sync_copy(dst4.at[s, p], didx)

        def step(i, carry):
            j0 = i * NBUF
            gs = [pltpu.async_copy(table.at[sidx.at[j0 + b]],
                                   rows.at[pl.ds(b * K, K)], semg)
                  for b in range(NBUF)]
            for g in gs:
                g.wait()
            ss = [pltpu.async_copy(rows.at[pl.ds(b * K, K)],
                                   acc.at[didx.at[j0 + b]], sems, add=True)
                  for b in range(NBUF)]
            for sc in ss:
                sc.wait()
            return carry

        lax.fori_loop(0, NCHS // NBUF, step, 0)

    plsc.subcore_barrier()
    pltpu.sync_copy(acc.at[pl.ds(r0, RPT)],
                    out.at[pl.ds(r0, RPT), pl.ds(c * DH, DH)])


@functools.partial(
    pl.kernel,
    out_type=jax.ShapeDtypeStruct((NC, NACC, D2), jnp.float32),
    mesh=_sc_mesh(),
    compiler_params=pltpu.CompilerParams(use_tc_tiling_on_sc=False),
    scratch_types=[
        pltpu.VMEM((NCH, K), jnp.int32),
        pltpu.VMEM((NCH, K), jnp.int32),
        pltpu.VMEM((8 * K, D2), jnp.float32),
        pltpu.VMEM_SHARED((NACC, D2), jnp.float32),
        pltpu.SemaphoreType.DMA,
        pltpu.SemaphoreType.DMA,
    ],
)
def _agg16(table, src3, dst3, out, sidx, didx, rows, acc, semg, sems):
    NBUF = 8
    c = lax.axis_index("c")
    s = lax.axis_index("s")
    wid = s * NC + c
    r0 = s * RPT

    idx_cp = pltpu.async_copy(src3.at[wid], sidx, semg)
    idx_cp2 = pltpu.async_copy(dst3.at[wid], didx, sems)

    def zrow(i, carry):
        rows[i, pl.ds(0, 16)] = jnp.zeros((16,), jnp.float32)
        return carry

    lax.fori_loop(0, K, zrow, 0)

    def zcp(t, carry):
        pltpu.sync_copy(rows.at[pl.ds(0, K)], acc.at[pl.ds(r0 + t * K, K)])
        return carry

    lax.fori_loop(0, RPT // K, zcp, 0)
    idx_cp.wait()
    idx_cp2.wait()
    plsc.subcore_barrier()

    def step(i, carry):
        j0 = i * NBUF
        gs = [pltpu.async_copy(table.at[sidx.at[j0 + b]],
                               rows.at[pl.ds(b * K, K)], semg)
              for b in range(NBUF)]
        for g in gs:
            g.wait()
        ss = [pltpu.async_copy(rows.at[pl.ds(b * K, K)],
                               acc.at[didx.at[j0 + b]], sems, add=True)
              for b in range(NBUF)]
        for sc in ss:
            sc.wait()
        return carry

    lax.fori_loop(0, NCH // NBUF, step, 0)
    plsc.subcore_barrier()
    pltpu.sync_copy(acc.at[pl.ds(r0, RPT)], out.at[c, pl.ds(r0, RPT)])


@functools.partial(
    pl.kernel,
    out_type=jax.ShapeDtypeStruct((NC, NACC), jnp.float32),
    mesh=_sc_mesh(),
    scratch_types=[
        pltpu.VMEM((NCH, K), jnp.int32),
        pltpu.VMEM((K,), jnp.float32),
        pltpu.VMEM((RPT,), jnp.float32),
        pltpu.VMEM_SHARED((NACC,), jnp.float32),
        pltpu.SemaphoreType.DMA,
        pltpu.SemaphoreType.DMA,
    ],
)
def _deg_kernel(dst3, out, didx, ones, zbuf, acc, semg, sems):
    NBUF = 8
    c = lax.axis_index("c")
    s = lax.axis_index("s")
    wid = s * NC + c
    r0 = s * RPT
    idx_cp = pltpu.async_copy(dst3.at[wid], didx, semg)
    for j in range(K // 16):
        ones[pl.ds(j * 16, 16)] = jnp.ones((16,), jnp.float32)

    def zfill(i, carry):
        zbuf[pl.ds(i * 16, 16)] = jnp.zeros((16,), jnp.float32)
        return carry

    lax.fori_loop(0, RPT // 16, zfill, 0)
    pltpu.sync_copy(zbuf, acc.at[pl.ds(r0, RPT)])
    idx_cp.wait()
    plsc.subcore_barrier()

    def step(i, carry):
        j0 = i * NBUF
        ss = [pltpu.async_copy(ones, acc.at[didx.at[j0 + b]], sems, add=True)
              for b in range(NBUF)]
        for sc in ss:
            sc.wait()
        return carry

    lax.fori_loop(0, NCH // NBUF, step, 0)
    plsc.subcore_barrier()
    pltpu.sync_copy(acc.at[pl.ds(r0, RPT)], out.at[c, pl.ds(r0, RPT)])


def _dinv_of(degb):
    d = degb[...]
    deg = d[:, 0:1] + d[:, 1:2] + 1.0
    return lax.rsqrt(deg)


def _dense1_body(xb, w, degb, ob):
    dinv = _dinv_of(degb)
    ob[...] = jnp.dot(xb[...], w[...], preferred_element_type=jnp.float32) * dinv


def _dense2_body(s1, h1s, degb, b1, w2p, ob):
    dinv = _dinv_of(degb)
    out1 = dinv * (s1[...] + h1s[...])
    a1 = jnp.maximum(out1 + b1[...], 0.0)
    ob[...] = jnp.dot(a1, w2p[...], preferred_element_type=jnp.float32) * dinv


def _final_body(s2a, s2b, h2s, degb, b2, ob):
    dinv = _dinv_of(degb)
    out2 = dinv * (s2a[...] + s2b[...] + h2s[...])
    bv = b2[...]
    za = out2[:, 0:1] + bv[:, 0:1]
    zb = out2[:, 1:2] + bv[:, 1:2]
    m = jnp.maximum(za, zb)
    lse = m + jnp.log(jnp.exp(za - m) + jnp.exp(zb - m))
    ob[...] = jnp.concatenate([za - lse, zb - lse], axis=1)


def _row_spec(d):
    return pl.BlockSpec((BM, d), lambda *g: (g[-1], 0))


def _full_spec(shape):
    return pl.BlockSpec(shape, lambda *g: tuple(0 for _ in shape))


_dense1 = pl.pallas_call(
    _dense1_body,
    grid=(N // BM,),
    in_specs=[_row_spec(D1), _full_spec((D1, D1)), _row_spec(2)],
    out_specs=_row_spec(D1),
    out_shape=jax.ShapeDtypeStruct((N, D1), jnp.float32),
)

_dense2 = pl.pallas_call(
    _dense2_body,
    grid=(N // BM,),
    in_specs=[_row_spec(D1), _row_spec(D1), _row_spec(2),
              _full_spec((1, D1)), _full_spec((D1, D2))],
    out_specs=_row_spec(D2),
    out_shape=jax.ShapeDtypeStruct((N, D2), jnp.float32),
)

_final = pl.pallas_call(
    _final_body,
    grid=(N // BM,),
    in_specs=[_row_spec(D2), _row_spec(D2), _row_spec(D2), _row_spec(2),
              _full_spec((1, 2))],
    out_specs=_row_spec(2),
    out_shape=jax.ShapeDtypeStruct((N, 2), jnp.float32),
)


def kernel(x, edge_index, W1, b1, W2, b2):
    src = edge_index[0].astype(jnp.int32)
    dst = edge_index[1].astype(jnp.int32)
    pad = EPAD - E
    srcp = jnp.concatenate([src, jnp.zeros((pad,), jnp.int32)])
    dstp = jnp.concatenate([dst, jnp.full((pad,), N, jnp.int32)])
    src3 = srcp.reshape(NW, NCH, K)
    dst3 = dstp.reshape(NW, NCH, K)
    src5 = jnp.stack([2 * srcp, 2 * srcp + 1]).reshape(NC, NS, NPASS, NCHS, K)
    dst4 = dstp.reshape(NS, NPASS, NCHS, K)

    degp = _deg_kernel(dst3)
    degt = jnp.transpose(degp)[:N]

    h1s = _dense1(x, W1, degt)
    s1 = _agg128(h1s.reshape(2 * N, DH), src5, dst4)

    W2p = jnp.concatenate(
        [W2, jnp.zeros((D1, D2 - W2.shape[1]), jnp.float32)], axis=1)
    h2s = _dense2(s1[:N], h1s, degt,
                  b1.reshape(1, D1), W2p)
    s2 = _agg16(h2s, src3, dst3)

    out = _final(s2[0, :N], s2[1, :N], h2s, degt, b2.reshape(1, 2))
    return out


# speedup vs baseline: 16.9605x; 1.0589x over previous
import functools

import jax
import jax.numpy as jnp
from jax import lax
from jax.experimental import pallas as pl
from jax.experimental.pallas import tpu as pltpu
from jax.experimental.pallas import tpu_sc as plsc

N = 10000
E = 320000
D1 = 128
DH = 64
D2 = 16
NC = 2
NS = 16
NW = NC * NS
EW = 10240
EPAD = EW * NW
K = 128
NCH = EW // K
EWS = EPAD // NS
NPASS = 2
NCHS = EWS // (K * NPASS)
NACC = 10240
RPT = NACC // NS
BM = 1000


def _sc_mesh():
    return plsc.VectorSubcoreMesh(core_axis_name="c", subcore_axis_name="s")


@functools.partial(
    pl.kernel,
    out_type=pltpu.HBM((NACC, D1), jnp.float32),
    mesh=_sc_mesh(),
    compiler_params=pltpu.CompilerParams(use_tc_tiling_on_sc=False),
    scratch_types=[
        pltpu.VMEM((NCHS, K), jnp.int32),
        pltpu.VMEM((NCHS, K), jnp.int32),
        pltpu.VMEM((8 * K, DH), jnp.float32),
        pltpu.VMEM_SHARED((NACC, DH), jnp.float32),
        pltpu.SemaphoreType.DMA,
        pltpu.SemaphoreType.DMA,
    ],
)
def _agg128(table, src5, dst4, out, sidx, didx, rows, acc, semg, sems):
    NBUF = 8
    c = lax.axis_index("c")
    s = lax.axis_index("s")
    r0 = s * RPT

    def zrow(i, carry):
        for j in range(DH // 16):
            rows[i, pl.ds(j * 16, 16)] = jnp.zeros((16,), jnp.float32)
        return carry

    lax.fori_loop(0, K, zrow, 0)

    def zcp(t, carry):
        pltpu.sync_copy(rows.at[pl.ds(0, K)], acc.at[pl.ds(r0 + t * K, K)])
        return carry

    lax.fori_loop(0, RPT // K, zcp, 0)
    plsc.subcore_barrier()

    for p in range(NPASS):
        pltpu.sync_copy(src5.at[c, s, p], sidx)
        pltpu.sync_copy(dst4.at[s, p], didx)

        def step(i, carry):
            j0 = i * NBUF
            gs = [pltpu.async_copy(table.at[sidx.at[j0 + b]],
                                   rows.at[pl.ds(b * K, K)], semg)
                  for b in range(NBUF)]
            ss = []
            for b in range(NBUF):
                gs[b].wait()
                ss.append(pltpu.async_copy(rows.at[pl.ds(b * K, K)],
                                           acc.at[didx.at[j0 + b]], sems,
                                           add=True))
            for sc in ss:
                sc.wait()
            return carry

        lax.fori_loop(0, NCHS // NBUF, step, 0)

    plsc.subcore_barrier()
    pltpu.sync_copy(acc.at[pl.ds(r0, RPT)],
                    out.at[pl.ds(r0, RPT), pl.ds(c * DH, DH)])


@functools.partial(
    pl.kernel,
    out_type=pltpu.HBM((NC, NACC, D2), jnp.float32),
    mesh=_sc_mesh(),
    compiler_params=pltpu.CompilerParams(use_tc_tiling_on_sc=False),
    scratch_types=[
        pltpu.VMEM((NCH, K), jnp.int32),
        pltpu.VMEM((NCH, K), jnp.int32),
        pltpu.VMEM((8 * K, D2), jnp.float32),
        pltpu.VMEM_SHARED((NACC, D2), jnp.float32),
        pltpu.SemaphoreType.DMA,
        pltpu.SemaphoreType.DMA,
    ],
)
def _agg16(table, src3, dst3, out, sidx, didx, rows, acc, semg, sems):
    NBUF = 8
    c = lax.axis_index("c")
    s = lax.axis_index("s")
    wid = s * NC + c
    r0 = s * RPT

    idx_cp = pltpu.async_copy(src3.at[wid], sidx, semg)
    idx_cp2 = pltpu.async_copy(dst3.at[wid], didx, sems)

    def zrow(i, carry):
        rows[i, pl.ds(0, 16)] = jnp.zeros((16,), jnp.float32)
        return carry

    lax.fori_loop(0, K, zrow, 0)

    def zcp(t, carry):
        pltpu.sync_copy(rows.at[pl.ds(0, K)], acc.at[pl.ds(r0 + t * K, K)])
        return carry

    lax.fori_loop(0, RPT // K, zcp, 0)
    idx_cp.wait()
    idx_cp2.wait()
    plsc.subcore_barrier()

    def step(i, carry):
        j0 = i * NBUF
        gs = [pltpu.async_copy(table.at[sidx.at[j0 + b]],
                               rows.at[pl.ds(b * K, K)], semg)
              for b in range(NBUF)]
        ss = []
        for b in range(NBUF):
            gs[b].wait()
            ss.append(pltpu.async_copy(rows.at[pl.ds(b * K, K)],
                                       acc.at[didx.at[j0 + b]], sems,
                                       add=True))
        for sc in ss:
            sc.wait()
        return carry

    lax.fori_loop(0, NCH // NBUF, step, 0)
    plsc.subcore_barrier()
    pltpu.sync_copy(acc.at[pl.ds(r0, RPT)], out.at[c, pl.ds(r0, RPT)])


@functools.partial(
    pl.kernel,
    out_type=pltpu.HBM((NC, NACC), jnp.float32),
    mesh=_sc_mesh(),
    scratch_types=[
        pltpu.VMEM((NCH, K), jnp.int32),
        pltpu.VMEM((K,), jnp.float32),
        pltpu.VMEM((RPT,), jnp.float32),
        pltpu.VMEM_SHARED((NACC,), jnp.float32),
        pltpu.SemaphoreType.DMA,
        pltpu.SemaphoreType.DMA,
    ],
)
def _deg_kernel(dst3, out, didx, ones, zbuf, acc, semg, sems):
    NBUF = 8
    c = lax.axis_index("c")
    s = lax.axis_index("s")
    wid = s * NC + c
    r0 = s * RPT
    idx_cp = pltpu.async_copy(dst3.at[wid], didx, semg)
    for j in range(K // 16):
        ones[pl.ds(j * 16, 16)] = jnp.ones((16,), jnp.float32)

    def zfill(i, carry):
        zbuf[pl.ds(i * 16, 16)] = jnp.zeros((16,), jnp.float32)
        return carry

    lax.fori_loop(0, RPT // 16, zfill, 0)
    pltpu.sync_copy(zbuf, acc.at[pl.ds(r0, RPT)])
    idx_cp.wait()
    plsc.subcore_barrier()

    def step(i, carry):
        j0 = i * NBUF
        ss = [pltpu.async_copy(ones, acc.at[didx.at[j0 + b]], sems, add=True)
              for b in range(NBUF)]
        for sc in ss:
            sc.wait()
        return carry

    lax.fori_loop(0, NCH // NBUF, step, 0)
    plsc.subcore_barrier()
    pltpu.sync_copy(acc.at[pl.ds(r0, RPT)], out.at[c, pl.ds(r0, RPT)])


def _dinv_of(degb):
    d = degb[...]
    deg = d[:, 0:1] + d[:, 1:2] + 1.0
    return lax.rsqrt(deg)


def _dense1_body(xb, w, degb, ob):
    dinv = _dinv_of(degb)
    ob[...] = jnp.dot(xb[...], w[...], preferred_element_type=jnp.float32) * dinv


def _dense2_body(s1, h1s, degb, b1, w2p, ob):
    dinv = _dinv_of(degb)
    out1 = dinv * (s1[...] + h1s[...])
    a1 = jnp.maximum(out1 + b1[...], 0.0)
    ob[...] = jnp.dot(a1, w2p[...], preferred_element_type=jnp.float32) * dinv


def _final_body(s2a, s2b, h2s, degb, b2, ob):
    dinv = _dinv_of(degb)
    out2 = dinv * (s2a[...] + s2b[...] + h2s[...])
    bv = b2[...]
    za = out2[:, 0:1] + bv[:, 0:1]
    zb = out2[:, 1:2] + bv[:, 1:2]
    m = jnp.maximum(za, zb)
    lse = m + jnp.log(jnp.exp(za - m) + jnp.exp(zb - m))
    ob[...] = jnp.concatenate([za - lse, zb - lse], axis=1)


def _row_spec(d):
    return pl.BlockSpec((BM, d), lambda *g: (g[-1], 0))


def _full_spec(shape):
    return pl.BlockSpec(shape, lambda *g: tuple(0 for _ in shape))


_dense1 = pl.pallas_call(
    _dense1_body,
    grid=(N // BM,),
    in_specs=[_row_spec(D1), _full_spec((D1, D1)), _row_spec(2)],
    out_specs=_row_spec(D1),
    out_shape=jax.ShapeDtypeStruct((N, D1), jnp.float32),
)

_dense2 = pl.pallas_call(
    _dense2_body,
    grid=(N // BM,),
    in_specs=[_row_spec(D1), _row_spec(D1), _row_spec(2),
              _full_spec((1, D1)), _full_spec((D1, D2))],
    out_specs=_row_spec(D2),
    out_shape=jax.ShapeDtypeStruct((N, D2), jnp.float32),
)

_final = pl.pallas_call(
    _final_body,
    grid=(N // BM,),
    in_specs=[_row_spec(D2), _row_spec(D2), _row_spec(D2), _row_spec(2),
              _full_spec((1, 2))],
    out_specs=_row_spec(2),
    out_shape=jax.ShapeDtypeStruct((N, 2), jnp.float32),
)


def kernel(x, edge_index, W1, b1, W2, b2):
    src = edge_index[0].astype(jnp.int32)
    dst = edge_index[1].astype(jnp.int32)
    pad = EPAD - E
    srcp = jnp.concatenate([src, jnp.zeros((pad,), jnp.int32)])
    dstp = jnp.concatenate([dst, jnp.full((pad,), N, jnp.int32)])
    src3 = srcp.reshape(NW, NCH, K)
    dst3 = dstp.reshape(NW, NCH, K)
    src5 = jnp.stack([2 * srcp, 2 * srcp + 1]).reshape(NC, NS, NPASS, NCHS, K)
    dst4 = dstp.reshape(NS, NPASS, NCHS, K)

    degp = _deg_kernel(dst3)
    degt = jnp.transpose(degp)[:N]

    h1s = _dense1(x, W1, degt)
    s1 = _agg128(h1s.reshape(2 * N, DH), src5, dst4)

    W2p = jnp.concatenate(
        [W2, jnp.zeros((D1, D2 - W2.shape[1]), jnp.float32)], axis=1)
    h2s = _dense2(s1[:N], h1s, degt,
                  b1.reshape(1, D1), W2p)
    s2 = _agg16(h2s, src3, dst3)

    out = _final(s2[0, :N], s2[1, :N], h2s, degt, b2.reshape(1, 2))
    return out


# speedup vs baseline: 39.5500x; 2.3319x over previous
import functools

import jax
import jax.numpy as jnp
from jax import lax
from jax.experimental import pallas as pl
from jax.experimental.pallas import tpu as pltpu
from jax.experimental.pallas import tpu_sc as plsc

N = 10000
E = 320000
D1 = 128
DH = 64
D2 = 16
NC = 2
NS = 16
NW = NC * NS
EW = 10240
EPAD = EW * NW
K = 128
NCH = EW // K
EWS = EPAD // NS
NPASS = 2
NCHS = EWS // (K * NPASS)
NACC = 10240
RPT = NACC // NS
BM = 1000


def _sc_mesh():
    return plsc.VectorSubcoreMesh(core_axis_name="c", subcore_axis_name="s")


@functools.partial(
    pl.kernel,
    out_type=pltpu.HBM((NACC, D1), jnp.float32),
    mesh=_sc_mesh(),
    compiler_params=pltpu.CompilerParams(use_tc_tiling_on_sc=False),
    scratch_types=[
        pltpu.VMEM((NCHS, K), jnp.int32),
        pltpu.VMEM((NCHS, K), jnp.int32),
        pltpu.VMEM((8 * K, DH), jnp.float32),
        pltpu.VMEM_SHARED((NACC, DH), jnp.float32),
        pltpu.SemaphoreType.DMA,
        pltpu.SemaphoreType.DMA,
    ],
)
def _agg128(table, src5, dst4, out, sidx, didx, rows, acc, semg, sems):
    NBUF = 8
    c = lax.axis_index("c")
    s = lax.axis_index("s")
    r0 = s * RPT

    def zrow(i, carry):
        for j in range(DH // 16):
            rows[i, pl.ds(j * 16, 16)] = jnp.zeros((16,), jnp.float32)
        return carry

    lax.fori_loop(0, K, zrow, 0)

    def zcp(t, carry):
        pltpu.sync_copy(rows.at[pl.ds(0, K)], acc.at[pl.ds(r0 + t * K, K)])
        return carry

    lax.fori_loop(0, RPT // K, zcp, 0)
    plsc.subcore_barrier()

    for p in range(NPASS):
        pltpu.sync_copy(src5.at[c, s, p], sidx)
        pltpu.sync_copy(dst4.at[s, p], didx)

        def step(i, carry):
            j0 = i * NBUF
            gs = [pltpu.async_copy(table.at[sidx.at[j0 + b]],
                                   rows.at[pl.ds(b * K, K)], semg)
                  for b in range(NBUF)]
            ss = []
            for b in range(NBUF):
                gs[b].wait()
                ss.append(pltpu.async_copy(rows.at[pl.ds(b * K, K)],
                                           acc.at[didx.at[j0 + b]], sems,
                                           add=True))
            for sc in ss:
                sc.wait()
            return carry

        lax.fori_loop(0, NCHS // NBUF, step, 0)

    plsc.subcore_barrier()
    pltpu.sync_copy(acc.at[pl.ds(r0, RPT)],
                    out.at[pl.ds(r0, RPT), pl.ds(c * DH, DH)])


@functools.partial(
    pl.kernel,
    out_type=pltpu.HBM((NC, NACC, D2), jnp.float32),
    mesh=_sc_mesh(),
    compiler_params=pltpu.CompilerParams(use_tc_tiling_on_sc=False),
    scratch_types=[
        pltpu.VMEM((NCH, K), jnp.int32),
        pltpu.VMEM((NCH, K), jnp.int32),
        pltpu.VMEM((10 * K, D2), jnp.float32),
        pltpu.VMEM_SHARED((NACC, D2), jnp.float32),
        pltpu.SemaphoreType.DMA,
        pltpu.SemaphoreType.DMA,
    ],
)
def _agg16(table, src3, dst3, out, sidx, didx, rows, acc, semg, sems):
    NBUF = 10
    c = lax.axis_index("c")
    s = lax.axis_index("s")
    wid = s * NC + c
    r0 = s * RPT

    idx_cp = pltpu.async_copy(src3.at[wid], sidx, semg)
    idx_cp2 = pltpu.async_copy(dst3.at[wid], didx, sems)

    def zrow(i, carry):
        rows[i, pl.ds(0, 16)] = jnp.zeros((16,), jnp.float32)
        return carry

    lax.fori_loop(0, K, zrow, 0)

    def zcp(t, carry):
        pltpu.sync_copy(rows.at[pl.ds(0, K)], acc.at[pl.ds(r0 + t * K, K)])
        return carry

    lax.fori_loop(0, RPT // K, zcp, 0)
    idx_cp.wait()
    idx_cp2.wait()
    plsc.subcore_barrier()

    def step(i, carry):
        j0 = i * NBUF
        gs = [pltpu.async_copy(table.at[sidx.at[j0 + b]],
                               rows.at[pl.ds(b * K, K)], semg)
              for b in range(NBUF)]
        ss = []
        for b in range(NBUF):
            gs[b].wait()
            ss.append(pltpu.async_copy(rows.at[pl.ds(b * K, K)],
                                       acc.at[didx.at[j0 + b]], sems,
                                       add=True))
        for sc in ss:
            sc.wait()
        return carry

    lax.fori_loop(0, NCH // NBUF, step, 0)
    plsc.subcore_barrier()
    pltpu.sync_copy(acc.at[pl.ds(r0, RPT)], out.at[c, pl.ds(r0, RPT)])


@functools.partial(
    pl.kernel,
    out_type=pltpu.HBM((NC, NACC), jnp.float32),
    mesh=_sc_mesh(),
    scratch_types=[
        pltpu.VMEM((NCH, K), jnp.int32),
        pltpu.VMEM((K,), jnp.float32),
        pltpu.VMEM((RPT,), jnp.float32),
        pltpu.VMEM_SHARED((NACC,), jnp.float32),
        pltpu.SemaphoreType.DMA,
        pltpu.SemaphoreType.DMA,
    ],
)
def _deg_kernel(dst3, out, didx, ones, zbuf, acc, semg, sems):
    NBUF = 8
    c = lax.axis_index("c")
    s = lax.axis_index("s")
    wid = s * NC + c
    r0 = s * RPT
    idx_cp = pltpu.async_copy(dst3.at[wid], didx, semg)
    for j in range(K // 16):
        ones[pl.ds(j * 16, 16)] = jnp.ones((16,), jnp.float32)

    def zfill(i, carry):
        zbuf[pl.ds(i * 16, 16)] = jnp.zeros((16,), jnp.float32)
        return carry

    lax.fori_loop(0, RPT // 16, zfill, 0)
    pltpu.sync_copy(zbuf, acc.at[pl.ds(r0, RPT)])
    idx_cp.wait()
    plsc.subcore_barrier()

    def step(i, carry):
        j0 = i * NBUF
        ss = [pltpu.async_copy(ones, acc.at[didx.at[j0 + b]], sems, add=True)
              for b in range(NBUF)]
        for sc in ss:
            sc.wait()
        return carry

    lax.fori_loop(0, NCH // NBUF, step, 0)
    plsc.subcore_barrier()
    pltpu.sync_copy(acc.at[pl.ds(r0, RPT)], out.at[c, pl.ds(r0, RPT)])


def _dinv_of(degb):
    d = degb[...]
    deg = d[:, 0:1] + d[:, 1:2] + 1.0
    return lax.rsqrt(deg)


def _dense1_body(xb, w, degb, ob):
    dinv = _dinv_of(degb)
    ob[...] = jnp.dot(xb[...], w[...], preferred_element_type=jnp.float32) * dinv


def _dense2_body(s1, h1s, degb, b1, w2p, ob):
    dinv = _dinv_of(degb)
    out1 = dinv * (s1[...] + h1s[...])
    a1 = jnp.maximum(out1 + b1[...], 0.0)
    ob[...] = jnp.dot(a1, w2p[...], preferred_element_type=jnp.float32) * dinv


def _final_body(s2a, s2b, h2s, degb, b2, ob):
    dinv = _dinv_of(degb)
    out2 = dinv * (s2a[...] + s2b[...] + h2s[...])
    bv = b2[...]
    za = out2[:, 0:1] + bv[:, 0:1]
    zb = out2[:, 1:2] + bv[:, 1:2]
    m = jnp.maximum(za, zb)
    lse = m + jnp.log(jnp.exp(za - m) + jnp.exp(zb - m))
    ob[...] = jnp.concatenate([za - lse, zb - lse], axis=1)


def _row_spec(d):
    return pl.BlockSpec((BM, d), lambda *g: (g[-1], 0))


def _full_spec(shape):
    return pl.BlockSpec(shape, lambda *g: tuple(0 for _ in shape))


_dense1 = pl.pallas_call(
    _dense1_body,
    grid=(N // BM,),
    in_specs=[_row_spec(D1), _full_spec((D1, D1)), _row_spec(2)],
    out_specs=_row_spec(D1),
    out_shape=jax.ShapeDtypeStruct((N, D1), jnp.float32),
)

_dense2 = pl.pallas_call(
    _dense2_body,
    grid=(N // BM,),
    in_specs=[_row_spec(D1), _row_spec(D1), _row_spec(2),
              _full_spec((1, D1)), _full_spec((D1, D2))],
    out_specs=_row_spec(D2),
    out_shape=jax.ShapeDtypeStruct((N, D2), jnp.float32),
)

_final = pl.pallas_call(
    _final_body,
    grid=(N // BM,),
    in_specs=[_row_spec(D2), _row_spec(D2), _row_spec(D2), _row_spec(2),
              _full_spec((1, 2))],
    out_specs=_row_spec(2),
    out_shape=jax.ShapeDtypeStruct((N, 2), jnp.float32),
)


def kernel(x, edge_index, W1, b1, W2, b2):
    src = edge_index[0].astype(jnp.int32)
    dst = edge_index[1].astype(jnp.int32)
    pad = EPAD - E
    pidx = jnp.arange(pad, dtype=jnp.int32)
    srcp = jnp.concatenate([src, pidx % N])
    dstp = jnp.concatenate([dst, N + pidx % (NACC - N)])
    src3 = srcp.reshape(NW, NCH, K)
    dst3 = dstp.reshape(NW, NCH, K)
    src5 = jnp.stack([2 * srcp, 2 * srcp + 1]).reshape(NC, NS, NPASS, NCHS, K)
    dst4 = dstp.reshape(NS, NPASS, NCHS, K)

    degp = _deg_kernel(dst3)
    degt = jnp.transpose(degp)[:N]

    h1s = _dense1(x, W1, degt)
    s1 = _agg128(h1s.reshape(2 * N, DH), src5, dst4)

    W2p = jnp.concatenate(
        [W2, jnp.zeros((D1, D2 - W2.shape[1]), jnp.float32)], axis=1)
    h2s = _dense2(s1[:N], h1s, degt,
                  b1.reshape(1, D1), W2p)
    s2 = _agg16(h2s, src3, dst3)

    out = _final(s2[0, :N], s2[1, :N], h2s, degt, b2.reshape(1, 2))
    return out


# speedup vs baseline: 39.6250x; 1.0019x over previous
import functools

import jax
import jax.numpy as jnp
from jax import lax
from jax.experimental import pallas as pl
from jax.experimental.pallas import tpu as pltpu
from jax.experimental.pallas import tpu_sc as plsc

N = 10000
E = 320000
D1 = 128
DH = 64
D2 = 16
NC = 2
NS = 16
NW = NC * NS
EW = E // NW
K = 125
NCH = EW // K
EWS = E // NS
NPASS = 2
NCHS = EWS // (K * NPASS)
NACC = 10240
RPT = NACC // NS
KZ = 128
BM = 1000


def _sc_mesh():
    return plsc.VectorSubcoreMesh(core_axis_name="c", subcore_axis_name="s")


@functools.partial(
    pl.kernel,
    out_type=pltpu.HBM((NACC, D1), jnp.float32),
    mesh=_sc_mesh(),
    compiler_params=pltpu.CompilerParams(use_tc_tiling_on_sc=False),
    scratch_types=[
        pltpu.VMEM((NCHS, K), jnp.int32),
        pltpu.VMEM((NCHS, K), jnp.int32),
        pltpu.VMEM((8 * K, DH), jnp.float32),
        pltpu.VMEM_SHARED((NACC, DH), jnp.float32),
        pltpu.SemaphoreType.DMA,
        pltpu.SemaphoreType.DMA,
    ],
)
def _agg128(table, src5, dst4, out, sidx, didx, rows, acc, semg, sems):
    NBUF = 8
    c = lax.axis_index("c")
    s = lax.axis_index("s")
    r0 = s * RPT

    def zrow(i, carry):
        for j in range(DH // 16):
            rows[i, pl.ds(j * 16, 16)] = jnp.zeros((16,), jnp.float32)
        return carry

    lax.fori_loop(0, KZ, zrow, 0)

    def zcp(t, carry):
        pltpu.sync_copy(rows.at[pl.ds(0, KZ)], acc.at[pl.ds(r0 + t * KZ, KZ)])
        return carry

    lax.fori_loop(0, RPT // KZ, zcp, 0)
    plsc.subcore_barrier()

    for p in range(NPASS):
        pltpu.sync_copy(src5.at[c, s, p], sidx)
        pltpu.sync_copy(dst4.at[s, p], didx)

        def step(i, carry):
            j0 = i * NBUF
            gs = [pltpu.async_copy(table.at[sidx.at[j0 + b]],
                                   rows.at[pl.ds(b * K, K)], semg)
                  for b in range(NBUF)]
            ss = []
            for b in range(NBUF):
                gs[b].wait()
                ss.append(pltpu.async_copy(rows.at[pl.ds(b * K, K)],
                                           acc.at[didx.at[j0 + b]], sems,
                                           add=True))
            for sc in ss:
                sc.wait()
            return carry

        lax.fori_loop(0, NCHS // NBUF, step, 0)

    plsc.subcore_barrier()
    pltpu.sync_copy(acc.at[pl.ds(r0, RPT)],
                    out.at[pl.ds(r0, RPT), pl.ds(c * DH, DH)])


@functools.partial(
    pl.kernel,
    out_type=pltpu.HBM((NC, NACC, D2), jnp.float32),
    mesh=_sc_mesh(),
    compiler_params=pltpu.CompilerParams(use_tc_tiling_on_sc=False),
    scratch_types=[
        pltpu.VMEM((NCH, K), jnp.int32),
        pltpu.VMEM((NCH, K), jnp.int32),
        pltpu.VMEM((10 * K, D2), jnp.float32),
        pltpu.VMEM_SHARED((NACC, D2), jnp.float32),
        pltpu.SemaphoreType.DMA,
        pltpu.SemaphoreType.DMA,
    ],
)
def _agg16(table, src3, dst3, out, sidx, didx, rows, acc, semg, sems):
    NBUF = 10
    c = lax.axis_index("c")
    s = lax.axis_index("s")
    wid = s * NC + c
    r0 = s * RPT

    idx_cp = pltpu.async_copy(src3.at[wid], sidx, semg)
    idx_cp2 = pltpu.async_copy(dst3.at[wid], didx, sems)

    def zrow(i, carry):
        rows[i, pl.ds(0, 16)] = jnp.zeros((16,), jnp.float32)
        return carry

    lax.fori_loop(0, KZ, zrow, 0)

    def zcp(t, carry):
        pltpu.sync_copy(rows.at[pl.ds(0, KZ)], acc.at[pl.ds(r0 + t * KZ, KZ)])
        return carry

    lax.fori_loop(0, RPT // KZ, zcp, 0)
    idx_cp.wait()
    idx_cp2.wait()
    plsc.subcore_barrier()

    def step(i, carry):
        j0 = i * NBUF
        gs = [pltpu.async_copy(table.at[sidx.at[j0 + b]],
                               rows.at[pl.ds(b * K, K)], semg)
              for b in range(NBUF)]
        ss = []
        for b in range(NBUF):
            gs[b].wait()
            ss.append(pltpu.async_copy(rows.at[pl.ds(b * K, K)],
                                       acc.at[didx.at[j0 + b]], sems,
                                       add=True))
        for sc in ss:
            sc.wait()
        return carry

    lax.fori_loop(0, NCH // NBUF, step, 0)
    plsc.subcore_barrier()
    pltpu.sync_copy(acc.at[pl.ds(r0, RPT)], out.at[c, pl.ds(r0, RPT)])


NCHD = E // (NS * K)


@functools.partial(
    pl.kernel,
    out_type=pltpu.HBM((NACC,), jnp.float32),
    mesh=_sc_mesh(),
    compiler_params=pltpu.CompilerParams(use_tc_tiling_on_sc=False),
    scratch_types=[
        pltpu.VMEM((NCHD, K), jnp.int32),
        pltpu.VMEM((K,), jnp.float32),
        pltpu.VMEM((RPT,), jnp.float32),
        pltpu.VMEM_SHARED((NACC,), jnp.float32),
        pltpu.SemaphoreType.DMA,
        pltpu.SemaphoreType.DMA,
    ],
)
def _deg_kernel(dst2, ones_in, out, didx, ones, zbuf, acc, semg, sems):
    NBUF = 8
    c = lax.axis_index("c")
    s = lax.axis_index("s")
    r0 = s * RPT

    @pl.when(c == 0)
    def _():
        idx_cp = pltpu.async_copy(dst2.at[s], didx, semg)
        pltpu.sync_copy(ones_in, ones)

        def zfill(i, carry):
            zbuf[pl.ds(i * 16, 16)] = jnp.zeros((16,), jnp.float32)
            return carry

        lax.fori_loop(0, RPT // 16, zfill, 0)
        pltpu.sync_copy(zbuf, acc.at[pl.ds(r0, RPT)])
        idx_cp.wait()
        plsc.subcore_barrier()

        def step(i, carry):
            j0 = i * NBUF
            ss = [pltpu.async_copy(ones, acc.at[didx.at[j0 + b]], sems,
                                   add=True)
                  for b in range(NBUF)]
            for sc in ss:
                sc.wait()
            return carry

        lax.fori_loop(0, NCHD // NBUF, step, 0)
        plsc.subcore_barrier()
        pltpu.sync_copy(acc.at[pl.ds(r0, RPT)], out.at[pl.ds(r0, RPT)])


def _dinv_of(degb):
    deg = degb[...] + 1.0
    return lax.rsqrt(deg)


def _dense1_body(xb, w, degb, ob):
    dinv = _dinv_of(degb)
    ob[...] = jnp.dot(xb[...], w[...], preferred_element_type=jnp.float32) * dinv


def _dense2_body(s1, h1s, degb, b1, w2p, ob):
    dinv = _dinv_of(degb)
    out1 = dinv * (s1[...] + h1s[...])
    a1 = jnp.maximum(out1 + b1[...], 0.0)
    ob[...] = jnp.dot(a1, w2p[...], preferred_element_type=jnp.float32) * dinv


def _final_body(s2a, s2b, h2s, degb, b2, ob):
    dinv = _dinv_of(degb)
    out2 = dinv * (s2a[...] + s2b[...] + h2s[...])
    bv = b2[...]
    za = out2[:, 0:1] + bv[:, 0:1]
    zb = out2[:, 1:2] + bv[:, 1:2]
    m = jnp.maximum(za, zb)
    lse = m + jnp.log(jnp.exp(za - m) + jnp.exp(zb - m))
    ob[...] = jnp.concatenate([za - lse, zb - lse], axis=1)


def _row_spec(d):
    return pl.BlockSpec((BM, d), lambda *g: (g[-1], 0))


def _full_spec(shape):
    return pl.BlockSpec(shape, lambda *g: tuple(0 for _ in shape))


_dense1 = pl.pallas_call(
    _dense1_body,
    grid=(N // BM,),
    in_specs=[_row_spec(D1), _full_spec((D1, D1)), _row_spec(1)],
    out_specs=_row_spec(D1),
    out_shape=jax.ShapeDtypeStruct((N, D1), jnp.float32),
)

_dense2 = pl.pallas_call(
    _dense2_body,
    grid=(N // BM,),
    in_specs=[_row_spec(D1), _row_spec(D1), _row_spec(1),
              _full_spec((1, D1)), _full_spec((D1, D2))],
    out_specs=_row_spec(D2),
    out_shape=jax.ShapeDtypeStruct((N, D2), jnp.float32),
)

_final = pl.pallas_call(
    _final_body,
    grid=(N // BM,),
    in_specs=[_row_spec(D2), _row_spec(D2), _row_spec(D2), _row_spec(1),
              _full_spec((1, 2))],
    out_specs=_row_spec(2),
    out_shape=jax.ShapeDtypeStruct((N, 2), jnp.float32),
)


def kernel(x, edge_index, W1, b1, W2, b2):
    src = edge_index[0].astype(jnp.int32)
    dst = edge_index[1].astype(jnp.int32)
    src3 = src.reshape(NW, NCH, K)
    dst3 = dst.reshape(NW, NCH, K)
    src5 = jnp.stack([2 * src, 2 * src + 1]).reshape(NC, NS, NPASS, NCHS, K)
    dst4 = dst.reshape(NS, NPASS, NCHS, K)
    dst2 = dst.reshape(NS, NCHD, K)

    deg = _deg_kernel(dst2, jnp.ones((K,), jnp.float32))
    degt = deg.reshape(NACC, 1)

    h1s = _dense1(x, W1, degt)
    s1 = _agg128(h1s.reshape(2 * N, DH), src5, dst4)

    W2p = jnp.concatenate(
        [W2, jnp.zeros((D1, D2 - W2.shape[1]), jnp.float32)], axis=1)
    h2s = _dense2(s1, h1s, degt,
                  b1.reshape(1, D1), W2p)
    s2 = _agg16(h2s, src3, dst3)

    out = _final(s2[0], s2[1], h2s, degt, b2.reshape(1, 2))
    return out


# speedup vs baseline: 39.8372x; 1.0054x over previous
import functools

import jax
import jax.numpy as jnp
from jax import lax
from jax.experimental import pallas as pl
from jax.experimental.pallas import tpu as pltpu
from jax.experimental.pallas import tpu_sc as plsc

N = 10000
E = 320000
D1 = 128
DH = 64
D2 = 16
NC = 2
NS = 16
NW = NC * NS
EW = E // NW
K = 125
NCH = EW // K
EWS = E // NS
NPASS = 4
NCHS = EWS // (K * NPASS)
NACC = 10240
RPT = NACC // NS
KZ = 128
BM = 1000


def _sc_mesh():
    return plsc.VectorSubcoreMesh(core_axis_name="c", subcore_axis_name="s")


@functools.partial(
    pl.kernel,
    out_type=pltpu.HBM((NACC, D1), jnp.float32),
    mesh=_sc_mesh(),
    compiler_params=pltpu.CompilerParams(use_tc_tiling_on_sc=False),
    scratch_types=[
        pltpu.VMEM((2, NCHS, K), jnp.int32),
        pltpu.VMEM((2, NCHS, K), jnp.int32),
        pltpu.VMEM((8 * K, DH), jnp.float32),
        pltpu.VMEM_SHARED((NACC, DH), jnp.float32),
        pltpu.SemaphoreType.DMA,
        pltpu.SemaphoreType.DMA,
        pltpu.SemaphoreType.DMA,
    ],
)
def _agg128(table, src5, dst4, out, sidx, didx, rows, acc, semg, sems, semi):
    NBUF = 8
    c = lax.axis_index("c")
    s = lax.axis_index("s")
    r0 = s * RPT

    icps = [pltpu.async_copy(src5.at[c, s, 0], sidx.at[0], semi),
            pltpu.async_copy(dst4.at[s, 0], didx.at[0], semi)]

    def zrow(i, carry):
        for j in range(DH // 16):
            rows[i, pl.ds(j * 16, 16)] = jnp.zeros((16,), jnp.float32)
        return carry

    lax.fori_loop(0, KZ, zrow, 0)

    def zcp(t, carry):
        pltpu.sync_copy(rows.at[pl.ds(0, KZ)], acc.at[pl.ds(r0 + t * KZ, KZ)])
        return carry

    lax.fori_loop(0, RPT // KZ, zcp, 0)
    plsc.subcore_barrier()

    for p in range(NPASS):
        for cp in icps:
            cp.wait()
        if p + 1 < NPASS:
            nxt = (p + 1) % 2
            icps = [pltpu.async_copy(src5.at[c, s, p + 1], sidx.at[nxt], semi),
                    pltpu.async_copy(dst4.at[s, p + 1], didx.at[nxt], semi)]
        else:
            icps = []
        cur = p % 2

        def step(i, carry):
            j0 = i * NBUF
            gs = [pltpu.async_copy(table.at[sidx.at[cur, j0 + b]],
                                   rows.at[pl.ds(b * K, K)], semg)
                  for b in range(NBUF)]
            ss = []
            for b in range(NBUF):
                gs[b].wait()
                ss.append(pltpu.async_copy(rows.at[pl.ds(b * K, K)],
                                           acc.at[didx.at[cur, j0 + b]], sems,
                                           add=True))
            for sc in ss:
                sc.wait()
            return carry

        lax.fori_loop(0, NCHS // NBUF, step, 0)

    plsc.subcore_barrier()
    pltpu.sync_copy(acc.at[pl.ds(r0, RPT)],
                    out.at[pl.ds(r0, RPT), pl.ds(c * DH, DH)])


@functools.partial(
    pl.kernel,
    out_type=pltpu.HBM((NC, NACC, D2), jnp.float32),
    mesh=_sc_mesh(),
    compiler_params=pltpu.CompilerParams(use_tc_tiling_on_sc=False),
    scratch_types=[
        pltpu.VMEM((NCH, K), jnp.int32),
        pltpu.VMEM((NCH, K), jnp.int32),
        pltpu.VMEM((10 * K, D2), jnp.float32),
        pltpu.VMEM_SHARED((NACC, D2), jnp.float32),
        pltpu.SemaphoreType.DMA,
        pltpu.SemaphoreType.DMA,
    ],
)
def _agg16(table, src3, dst3, out, sidx, didx, rows, acc, semg, sems):
    NBUF = 10
    c = lax.axis_index("c")
    s = lax.axis_index("s")
    wid = s * NC + c
    r0 = s * RPT

    idx_cp = pltpu.async_copy(src3.at[wid], sidx, semg)
    idx_cp2 = pltpu.async_copy(dst3.at[wid], didx, sems)

    def zrow(i, carry):
        rows[i, pl.ds(0, 16)] = jnp.zeros((16,), jnp.float32)
        return carry

    lax.fori_loop(0, KZ, zrow, 0)

    def zcp(t, carry):
        pltpu.sync_copy(rows.at[pl.ds(0, KZ)], acc.at[pl.ds(r0 + t * KZ, KZ)])
        return carry

    lax.fori_loop(0, RPT // KZ, zcp, 0)
    idx_cp.wait()
    idx_cp2.wait()
    plsc.subcore_barrier()

    def step(i, carry):
        j0 = i * NBUF
        gs = [pltpu.async_copy(table.at[sidx.at[j0 + b]],
                               rows.at[pl.ds(b * K, K)], semg)
              for b in range(NBUF)]
        ss = []
        for b in range(NBUF):
            gs[b].wait()
            ss.append(pltpu.async_copy(rows.at[pl.ds(b * K, K)],
                                       acc.at[didx.at[j0 + b]], sems,
                                       add=True))
        for sc in ss:
            sc.wait()
        return carry

    lax.fori_loop(0, NCH // NBUF, step, 0)
    plsc.subcore_barrier()
    pltpu.sync_copy(acc.at[pl.ds(r0, RPT)], out.at[c, pl.ds(r0, RPT)])


NCHD = E // (NS * K)


@functools.partial(
    pl.kernel,
    out_type=pltpu.HBM((NACC,), jnp.float32),
    mesh=_sc_mesh(),
    compiler_params=pltpu.CompilerParams(use_tc_tiling_on_sc=False),
    scratch_types=[
        pltpu.VMEM((NCHD, K), jnp.int32),
        pltpu.VMEM((K,), jnp.float32),
        pltpu.VMEM((RPT,), jnp.float32),
        pltpu.VMEM_SHARED((NACC,), jnp.float32),
        pltpu.SemaphoreType.DMA,
        pltpu.SemaphoreType.DMA,
    ],
)
def _deg_kernel(dst2, ones_in, out, didx, ones, zbuf, acc, semg, sems):
    NBUF = 8
    c = lax.axis_index("c")
    s = lax.axis_index("s")
    r0 = s * RPT

    @pl.when(c == 0)
    def _():
        idx_cp = pltpu.async_copy(dst2.at[s], didx, semg)
        pltpu.sync_copy(ones_in, ones)

        def zfill(i, carry):
            zbuf[pl.ds(i * 16, 16)] = jnp.zeros((16,), jnp.float32)
            return carry

        lax.fori_loop(0, RPT // 16, zfill, 0)
        pltpu.sync_copy(zbuf, acc.at[pl.ds(r0, RPT)])
        idx_cp.wait()
        plsc.subcore_barrier()

        def step(i, carry):
            j0 = i * NBUF
            ss = [pltpu.async_copy(ones, acc.at[didx.at[j0 + b]], sems,
                                   add=True)
                  for b in range(NBUF)]
            for sc in ss:
                sc.wait()
            return carry

        lax.fori_loop(0, NCHD // NBUF, step, 0)
        plsc.subcore_barrier()
        pltpu.sync_copy(acc.at[pl.ds(r0, RPT)], out.at[pl.ds(r0, RPT)])


def _dinv_of(degb):
    deg = degb[...] + 1.0
    return lax.rsqrt(deg)


def _dense1_body(xb, w, degb, ob):
    dinv = _dinv_of(degb)
    ob[...] = jnp.dot(xb[...], w[...], preferred_element_type=jnp.float32) * dinv


def _dense2_body(s1, h1s, degb, b1, w2p, ob):
    dinv = _dinv_of(degb)
    out1 = dinv * (s1[...] + h1s[...])
    a1 = jnp.maximum(out1 + b1[...], 0.0)
    ob[...] = jnp.dot(a1, w2p[...], preferred_element_type=jnp.float32) * dinv


def _final_body(s2a, s2b, h2s, degb, b2, ob):
    dinv = _dinv_of(degb)
    out2 = dinv * (s2a[...] + s2b[...] + h2s[...])
    bv = b2[...]
    za = out2[:, 0:1] + bv[:, 0:1]
    zb = out2[:, 1:2] + bv[:, 1:2]
    m = jnp.maximum(za, zb)
    lse = m + jnp.log(jnp.exp(za - m) + jnp.exp(zb - m))
    ob[...] = jnp.concatenate([za - lse, zb - lse], axis=1)


def _row_spec(d):
    return pl.BlockSpec((BM, d), lambda *g: (g[-1], 0))


def _full_spec(shape):
    return pl.BlockSpec(shape, lambda *g: tuple(0 for _ in shape))


_dense1 = pl.pallas_call(
    _dense1_body,
    grid=(N // BM,),
    in_specs=[_row_spec(D1), _full_spec((D1, D1)), _row_spec(1)],
    out_specs=_row_spec(D1),
    out_shape=jax.ShapeDtypeStruct((N, D1), jnp.float32),
)

_dense2 = pl.pallas_call(
    _dense2_body,
    grid=(N // BM,),
    in_specs=[_row_spec(D1), _row_spec(D1), _row_spec(1),
              _full_spec((1, D1)), _full_spec((D1, D2))],
    out_specs=_row_spec(D2),
    out_shape=jax.ShapeDtypeStruct((N, D2), jnp.float32),
)

_final = pl.pallas_call(
    _final_body,
    grid=(N // BM,),
    in_specs=[_row_spec(D2), _row_spec(D2), _row_spec(D2), _row_spec(1),
              _full_spec((1, 2))],
    out_specs=_row_spec(2),
    out_shape=jax.ShapeDtypeStruct((N, 2), jnp.float32),
)


def kernel(x, edge_index, W1, b1, W2, b2):
    src = edge_index[0].astype(jnp.int32)
    dst = edge_index[1].astype(jnp.int32)
    src3 = src.reshape(NW, NCH, K)
    dst3 = dst.reshape(NW, NCH, K)
    src5 = jnp.stack([2 * src, 2 * src + 1]).reshape(NC, NS, NPASS, NCHS, K)
    dst4 = dst.reshape(NS, NPASS, NCHS, K)
    dst2 = dst.reshape(NS, NCHD, K)

    deg = _deg_kernel(dst2, jnp.ones((K,), jnp.float32))
    degt = deg.reshape(NACC, 1)

    h1s = _dense1(x, W1, degt)
    s1 = _agg128(h1s.reshape(2 * N, DH), src5, dst4)

    W2p = jnp.concatenate(
        [W2, jnp.zeros((D1, D2 - W2.shape[1]), jnp.float32)], axis=1)
    h2s = _dense2(s1, h1s, degt,
                  b1.reshape(1, D1), W2p)
    s2 = _agg16(h2s, src3, dst3)

    out = _final(s2[0], s2[1], h2s, degt, b2.reshape(1, 2))
    return out
